# baseline (speedup 1.0000x reference)
import functools

import jax
import jax.numpy as jnp
from jax import lax
from jax.experimental import pallas as pl
from jax.experimental.pallas import tpu as pltpu
from jax.experimental.pallas import tpu_sc as plsc

_NSUB = 16
_LANES = 16


def _diag_gather(x2d, idx2):
    nrows, hw = x2d.shape
    rows_per_sub = nrows // _NSUB
    nchunk = 8
    chunk = hw // nchunk
    nvec = chunk // _LANES

    mesh = plsc.VectorSubcoreMesh(core_axis_name="c", subcore_axis_name="s")

    @functools.partial(
        pl.kernel,
        mesh=mesh,
        compiler_params=pltpu.CompilerParams(needs_layout_passes=False),
        out_type=jax.ShapeDtypeStruct((2, nrows, hw), jnp.float32),
        scratch_types=[
            pltpu.VMEM((hw,), jnp.int32),
            pltpu.VMEM((hw,), jnp.float32),
            pltpu.VMEM((chunk,), jnp.float32),
        ],
    )
    def k(x_hbm, idx_hbm, out_hbm, idx_v, row_v, ob):
        c = lax.axis_index("c")
        s = lax.axis_index("s")

        pltpu.sync_copy(idx_hbm.at[c], idx_v)

        def row_body(r, carry):
            row = s * rows_per_sub + r
            pltpu.sync_copy(x_hbm.at[row], row_v)

            def chunk_body(ch, carry2):
                @plsc.parallel_loop(0, chunk, step=_LANES, unroll=16)
                def _(v):
                    iv = idx_v[pl.ds(ch * chunk + v, _LANES)]
                    ob[pl.ds(v, _LANES)] = plsc.load_gather(row_v, [iv])

                pltpu.sync_copy(
                    ob, out_hbm.at[c, row, pl.ds(ch * chunk, chunk)])
                return carry2

            lax.fori_loop(0, nchunk, chunk_body, 0)
            return carry

        lax.fori_loop(0, rows_per_sub, row_body, 0)

    return k(x2d, idx2)


def kernel(x, rd_index, ld_index):
    B, C, H, W = x.shape
    x2d = x.reshape(B * C, H * W)
    idx2 = jnp.stack(
        [rd_index.astype(jnp.int32), ld_index.astype(jnp.int32)])
    out = _diag_gather(x2d, idx2)
    return out[0].reshape(B, C, H * W), out[1].reshape(B, C, H * W)


# speedup vs baseline: 1.0350x; 1.0350x over previous
import functools

import jax
import jax.numpy as jnp
from jax import lax
from jax.experimental import pallas as pl
from jax.experimental.pallas import tpu as pltpu
from jax.experimental.pallas import tpu_sc as plsc

_NSUB = 16
_LANES = 16
_NCHUNK = 8


def _diag_gather(x2d, idx2):
    nrows, hw = x2d.shape
    rows_per_sub = nrows // _NSUB
    chunk = hw // _NCHUNK
    assert chunk % _LANES == 0 and nrows % _NSUB == 0

    mesh = plsc.VectorSubcoreMesh(core_axis_name="c", subcore_axis_name="s")

    @functools.partial(
        pl.kernel,
        mesh=mesh,
        compiler_params=pltpu.CompilerParams(needs_layout_passes=False),
        out_type=jax.ShapeDtypeStruct((2, nrows, hw), jnp.float32),
        scratch_types=[
            pltpu.VMEM((hw,), jnp.int32),
            pltpu.VMEM((hw,), jnp.float32),
            pltpu.VMEM((chunk,), jnp.float32),
        ],
    )
    def k(x_hbm, idx_hbm, out_hbm, idx_v, rowv, obuf):
        c = lax.axis_index("c")
        s = lax.axis_index("s")
        base = s * rows_per_sub

        pltpu.sync_copy(idx_hbm.at[c], idx_v)

        def row_body(r, carry):
            row = base + r
            pltpu.sync_copy(x_hbm.at[row], rowv)

            def chunk_body(ch, carry2):
                @plsc.parallel_loop(0, chunk, step=_LANES, unroll=8)
                def _(v):
                    iv = idx_v[pl.ds(ch * chunk + v, _LANES)]
                    obuf[pl.ds(v, _LANES)] = plsc.load_gather(rowv, [iv])

                pltpu.sync_copy(
                    obuf, out_hbm.at[c, row, pl.ds(ch * chunk, chunk)])
                return carry2

            lax.fori_loop(0, _NCHUNK, chunk_body, 0)
            return carry

        lax.fori_loop(0, rows_per_sub, row_body, 0)

    return k(x2d, idx2)


def kernel(x, rd_index, ld_index):
    B, C, H, W = x.shape
    x2d = x.reshape(B * C, H * W)
    idx2 = jnp.stack([rd_index, ld_index]).astype(jnp.int32)
    out = _diag_gather(x2d, idx2)
    return out[0].reshape(B, C, H * W), out[1].reshape(B, C, H * W)
